# Initial kernel scaffold; baseline (speedup 1.0000x reference)
import jax
import jax.numpy as jnp
from jax.experimental import pallas as pl


def kernel(feature, edge_list, W1, a_src1, a_dst1, b1, W2, a_src2, a_dst2, b2, Wl, bl, Wp, bp):
    raise NotImplementedError("write your pallas kernel here")


# speedup vs baseline: 87.4485x; 87.4485x over previous
import functools

import jax
import jax.numpy as jnp
from jax import lax
from jax.experimental import pallas as pl
from jax.experimental.pallas import tpu as pltpu
from jax.experimental.pallas import tpu_sc as plsc

B, N, E = 1024, 14, 64
F_IN, HID, HEADS = 128, 256, 8
NP = 16
G = 16
R = G * NP
STEPS = B // G


_NC, _NS = 2, 16
_NW = _NC * _NS
_GPW = B // _NW
_CH = 8
_NCHUNK = _GPW // _CH


def _sc_counts_body(src_hbm, dst_hbm, out_hbm, src_v, dst_v, cnt_v):
    wid = lax.axis_index("s") * _NC + lax.axis_index("c")
    ones = jnp.ones((16,), jnp.float32)
    zeros = jnp.zeros((16,), jnp.float32)
    io = lax.iota(jnp.int32, 16)
    selfmask = io < N
    cwords = _CH * NP * NP
    for c in range(_NCHUNK):
        base = wid * _GPW + c * _CH
        pltpu.sync_copy(src_hbm.at[pl.ds(base * E, _CH * E)], src_v)
        pltpu.sync_copy(dst_hbm.at[pl.ds(base * E, _CH * E)], dst_v)
        for q in range(cwords // 16):
            cnt_v[pl.ds(q * 16, 16)] = zeros
        for g in range(_CH):
            goff = g * NP * NP
            for j in range(E // 16):
                s = src_v[pl.ds(g * E + j * 16, 16)]
                d = dst_v[pl.ds(g * E + j * 16, 16)]
                plsc.addupdate_scatter(cnt_v, [goff + d * NP + s], ones)
            plsc.addupdate_scatter(cnt_v, [goff + io * (NP + 1)], ones,
                                   mask=selfmask)
        pltpu.sync_copy(cnt_v, out_hbm.at[pl.ds(base * NP * NP, cwords)])


def _sc_counts(src32, dst32):
    mesh = plsc.VectorSubcoreMesh(core_axis_name="c", subcore_axis_name="s")
    fn = functools.partial(
        pl.kernel,
        mesh=mesh,
        out_type=jax.ShapeDtypeStruct((B * NP * NP,), jnp.float32),
        scratch_types=[
            pltpu.VMEM((_CH * E,), jnp.int32),
            pltpu.VMEM((_CH * E,), jnp.int32),
            pltpu.VMEM((_CH * NP * NP,), jnp.float32),
        ],
        compiler_params=pltpu.CompilerParams(needs_layout_passes=False),
    )(_sc_counts_body)
    return fn(src32.reshape(B * E), dst32.reshape(B * E))


def _dot(a, b):
    return jnp.dot(a, b, preferred_element_type=jnp.float32)


def _tc_body(x_ref, cnt_ref, w1_ref, av_ref, b1_ref, w2_ref, b2_ref,
             wl_ref, bl_ref, wpt_ref, bp_ref, out_ref):
    f32 = jnp.float32
    x = x_ref[...]
    cnt = cnt_ref[...]

    rr = lax.broadcasted_iota(jnp.int32, (R, R), 0)
    cc = lax.broadcasted_iota(jnp.int32, (R, R), 1)
    bsum = (rr // NP == cc // NP).astype(f32)
    perm = ((rr % G == cc // NP) & (rr // G == cc % NP)).astype(f32)
    r16 = lax.broadcasted_iota(jnp.int32, (NP, R), 0)
    c16 = lax.broadcasted_iota(jnp.int32, (NP, R), 1)
    tile16 = (c16 % NP == r16).astype(f32)
    rh = lax.broadcasted_iota(jnp.int32, (HEADS, 8 * NP), 0)
    ch = lax.broadcasted_iota(jnp.int32, (HEADS, 8 * NP), 1)
    tileh = (ch // NP == rh).astype(f32)
    re = lax.broadcasted_iota(jnp.int32, (NP, 8 * NP), 0)
    ce = lax.broadcasted_iota(jnp.int32, (NP, 8 * NP), 1)
    cexph = (ce % NP == re).astype(f32)
    rs = lax.broadcasted_iota(jnp.int32, (8 * NP, HEADS), 0)
    cs = lax.broadcasted_iota(jnp.int32, (8 * NP, HEADS), 1)
    sumh = (rs // NP == cs).astype(f32)
    rm = lax.broadcasted_iota(jnp.int32, (R, 8 * NP), 0)
    cm = lax.broadcasted_iota(jnp.int32, (R, 8 * NP), 1)
    ms = (rm % NP == cm % NP).astype(f32)

    c128 = _dot(cnt, cexph)
    cmask = c128 > 0.0

    def gat(xin, w, aoff, bvec):
        h = _dot(xin, w)
        asrc = _dot(h, av_ref[:, aoff:aoff + 8])
        adst = _dot(h, av_ref[:, aoff + 8:aoff + 16])
        dstp = _dot(adst, tileh)
        srcp = _dot(bsum, _dot(asrc, tileh) * ms)
        al = srcp + dstp
        al = jnp.where(al >= 0.0, al, 0.2 * al)
        mrow = jnp.max(jnp.where(cmask, al, -1e30), axis=1, keepdims=True)
        ee = jnp.where(cmask, c128 * jnp.exp(al - mrow), 0.0)
        ssum = _dot(_dot(ee, sumh), tileh)
        att = ee / (ssum + 1e-16)
        out = None
        for hh in range(HEADS):
            ah = att[:, hh * NP:(hh + 1) * NP]
            bd = _dot(ah, tile16) * bsum
            part = _dot(bd, h[:, hh * HID:(hh + 1) * HID])
            out = part if out is None else out + part
        return out * (1.0 / HEADS) + bvec[...]

    h1 = jnp.maximum(gat(x, w1_ref[...], 0, b1_ref), 0.0)
    h2 = gat(h1, w2_ref[...], 16, b2_ref)
    hp = _dot(perm, h2)
    zacc = None
    for d in range(N):
        part = _dot(hp[d * G:(d + 1) * G, :], wl_ref[d * HID:(d + 1) * HID, :])
        zacc = part if zacc is None else zacc + part
    z = zacc + bl_ref[...]
    logit = jnp.sum(z * wpt_ref[...], axis=1, keepdims=True) + bp_ref[...]
    out_ref[...] = 1.0 / (1.0 + jnp.exp(-logit))


def kernel(feature, edge_list, W1, a_src1, a_dst1, b1, W2, a_src2, a_dst2, b2,
           Wl, bl, Wp, bp):
    f32 = jnp.float32
    el = edge_list.astype(jnp.int32)
    src32 = el[:, :, 0]
    dst32 = el[:, :, 1]
    counts = _sc_counts(src32, dst32)
    cnt_rows = counts.reshape(B * NP, NP)

    xp = jnp.pad(feature, ((0, 0), (0, NP - N), (0, 0))).reshape(B * NP, F_IN)

    def mk_a(a):
        return (a[:, :, None] * jnp.eye(HEADS, dtype=f32)[:, None, :]) \
            .reshape(HEADS * HID, HEADS)

    av = jnp.concatenate(
        [mk_a(a_src1), mk_a(a_dst1), mk_a(a_src2), mk_a(a_dst2)], axis=1)

    grid = (STEPS,)
    out = pl.pallas_call(
        _tc_body,
        grid=grid,
        in_specs=[
            pl.BlockSpec((R, F_IN), lambda i: (i, 0)),
            pl.BlockSpec((R, NP), lambda i: (i, 0)),
            pl.BlockSpec((F_IN, HEADS * HID), lambda i: (0, 0)),
            pl.BlockSpec((HEADS * HID, 32), lambda i: (0, 0)),
            pl.BlockSpec((1, HID), lambda i: (0, 0)),
            pl.BlockSpec((HID, HEADS * HID), lambda i: (0, 0)),
            pl.BlockSpec((1, HID), lambda i: (0, 0)),
            pl.BlockSpec((N * HID, HID // 2), lambda i: (0, 0)),
            pl.BlockSpec((1, HID // 2), lambda i: (0, 0)),
            pl.BlockSpec((1, HID // 2), lambda i: (0, 0)),
            pl.BlockSpec((1, 1), lambda i: (0, 0)),
        ],
        out_specs=pl.BlockSpec((G, 1), lambda i: (i, 0)),
        out_shape=jax.ShapeDtypeStruct((B, 1), f32),
        compiler_params=pltpu.CompilerParams(
            dimension_semantics=("arbitrary",)),
    )(xp, cnt_rows, W1, av, b1.reshape(1, HID), W2, b2.reshape(1, HID),
      Wl, bl.reshape(1, HID // 2), Wp.reshape(1, HID // 2),
      bp.reshape(1, 1))
    return out


# speedup vs baseline: 97.0739x; 1.1101x over previous
import functools

import jax
import jax.numpy as jnp
from jax import lax
from jax.experimental import pallas as pl
from jax.experimental.pallas import tpu as pltpu
from jax.experimental.pallas import tpu_sc as plsc

B, N, E = 1024, 14, 64
F_IN, HID, HEADS = 128, 256, 8
NP = 16
G = 16
R = G * NP
VSUB = 2
STEPS = B // (G * VSUB)


_NC, _NS = 2, 16
_NW = _NC * _NS
_GPW = B // _NW
_CH = 8
_NCHUNK = _GPW // _CH


def _sc_counts_body(src_hbm, dst_hbm, out_hbm, src_v, dst_v, cnt_v):
    wid = lax.axis_index("s") * _NC + lax.axis_index("c")
    ones = jnp.ones((16,), jnp.float32)
    zeros = jnp.zeros((16,), jnp.float32)
    io = lax.iota(jnp.int32, 16)
    selfmask = io < N
    cwords = _CH * NP * NP
    for c in range(_NCHUNK):
        base = wid * _GPW + c * _CH
        pltpu.sync_copy(src_hbm.at[pl.ds(base * E, _CH * E)], src_v)
        pltpu.sync_copy(dst_hbm.at[pl.ds(base * E, _CH * E)], dst_v)
        for q in range(cwords // 16):
            cnt_v[pl.ds(q * 16, 16)] = zeros
        for g in range(_CH):
            goff = g * NP * NP
            for j in range(E // 16):
                s = src_v[pl.ds(g * E + j * 16, 16)]
                d = dst_v[pl.ds(g * E + j * 16, 16)]
                plsc.addupdate_scatter(cnt_v, [goff + d * NP + s], ones)
            plsc.addupdate_scatter(cnt_v, [goff + io * (NP + 1)], ones,
                                   mask=selfmask)
        pltpu.sync_copy(cnt_v, out_hbm.at[pl.ds(base * NP * NP, cwords)])


def _sc_counts(src32, dst32):
    mesh = plsc.VectorSubcoreMesh(core_axis_name="c", subcore_axis_name="s")
    fn = functools.partial(
        pl.kernel,
        mesh=mesh,
        out_type=jax.ShapeDtypeStruct((B * NP * NP,), jnp.float32),
        scratch_types=[
            pltpu.VMEM((_CH * E,), jnp.int32),
            pltpu.VMEM((_CH * E,), jnp.int32),
            pltpu.VMEM((_CH * NP * NP,), jnp.float32),
        ],
        compiler_params=pltpu.CompilerParams(needs_layout_passes=False),
    )(_sc_counts_body)
    return fn(src32.reshape(B * E), dst32.reshape(B * E))


def _dot(a, b):
    return jnp.dot(a, b, preferred_element_type=jnp.float32)


def _tc_body(x_ref, cnt_ref, w1_ref, aw1_ref, b1_ref, w2_ref, aw2_ref,
             b2_ref, wl_ref, bl_ref, wpt_ref, bp_ref,
             bsum_ref, permb_ref, msrc_ref, t16big_ref, tileh_ref, cexph_ref,
             sumh_ref, bmask8_ref, out_ref):
    bf16 = jnp.bfloat16
    bsum = bsum_ref[...]
    msrc = msrc_ref[...]
    tileh = tileh_ref[...]

    def gat(cmask, c128, xin, xf, w, aw_ref, bvec):
        h = _dot(xin, w[...])
        sa = _dot(xf, aw_ref[...])
        asrc = sa[:, :HEADS]
        adst = sa[:, HEADS:]
        dstp = _dot(adst, tileh)
        srcp = _dot(bsum, _dot(asrc, tileh) * msrc)
        al = srcp + dstp
        al = jnp.where(al >= 0.0, al, 0.2 * al)
        mrow = jnp.max(jnp.where(cmask, al, -1e30), axis=1, keepdims=True)
        ee = jnp.where(cmask, c128 * jnp.exp(al - mrow), 0.0)
        ssum = _dot(_dot(ee, sumh_ref[...]), tileh)
        att = (ee / (ssum + 1e-16)).astype(bf16)
        hb = h.astype(bf16)
        bdall = _dot(att, t16big_ref[...]).astype(bf16) * bmask8_ref[...]
        out = None
        for hh in range(HEADS):
            part = _dot(bdall[:, hh * R:(hh + 1) * R],
                        hb[:, hh * HID:(hh + 1) * HID])
            out = part if out is None else out + part
        return (out * (1.0 / HEADS) + bvec[...]).astype(bf16)

    wl = wl_ref[...]
    for v in range(VSUB):
        x = x_ref[v * R:(v + 1) * R, :]
        cnt = cnt_ref[v * R:(v + 1) * R, :]
        c128 = _dot(cnt, cexph_ref[...])
        cmask = c128 > 0.0
        h1 = jnp.maximum(
            gat(cmask, c128, x.astype(bf16), x, w1_ref, aw1_ref, b1_ref), 0.0)
        h2 = gat(cmask, c128, h1, h1.astype(jnp.float32), w2_ref, aw2_ref,
                 b2_ref)
        hp = _dot(permb_ref[...], h2).astype(bf16)
        zacc = None
        for d in range(N):
            part = _dot(hp[d * G:(d + 1) * G, :],
                        wl[d * HID:(d + 1) * HID, :])
            zacc = part if zacc is None else zacc + part
        z = zacc + bl_ref[...]
        logit = jnp.sum(z * wpt_ref[...], axis=1, keepdims=True) + bp_ref[...]
        out_ref[v * G:(v + 1) * G, :] = 1.0 / (1.0 + jnp.exp(-logit))


def _full(shape):
    return pl.BlockSpec(shape, lambda i: tuple(0 for _ in shape))


def kernel(feature, edge_list, W1, a_src1, a_dst1, b1, W2, a_src2, a_dst2, b2,
           Wl, bl, Wp, bp):
    f32 = jnp.float32
    bf16 = jnp.bfloat16
    el = edge_list.astype(jnp.int32)
    src32 = el[:, :, 0]
    dst32 = el[:, :, 1]
    counts = _sc_counts(src32, dst32)
    cnt_rows = counts.reshape(B * NP, NP)

    xp = jnp.pad(feature, ((0, 0), (0, NP - N), (0, 0))).reshape(B * NP, F_IN)

    def mk_a(a):
        return (a[:, :, None] * jnp.eye(HEADS, dtype=f32)[:, None, :]) \
            .reshape(HEADS * HID, HEADS)

    aw1 = jnp.dot(W1, jnp.concatenate([mk_a(a_src1), mk_a(a_dst1)], axis=1))
    aw2 = jnp.dot(W2, jnp.concatenate([mk_a(a_src2), mk_a(a_dst2)], axis=1))

    rr = jnp.arange(R)[:, None]
    cc = jnp.arange(R)[None, :]
    bsum = (rr // NP == cc // NP).astype(f32)
    permb = ((rr % G == cc // NP) & (rr // G == cc % NP)).astype(bf16)
    l128 = jnp.arange(8 * NP)[None, :]
    msrc = (rr[:, :1] % NP == l128 % NP).astype(f32)
    lbig = jnp.arange(HEADS * R)[None, :]
    t16big = ((jnp.arange(8 * NP)[:, None] // NP == lbig // R) &
              (jnp.arange(8 * NP)[:, None] % NP == lbig % NP)).astype(bf16)
    bmask8 = (rr // NP == (lbig % R) // NP).astype(bf16)
    tileh = (l128 // NP == jnp.arange(HEADS)[:, None]).astype(f32)
    cexph = (l128 % NP == jnp.arange(NP)[:, None]).astype(f32)
    sumh = (jnp.arange(8 * NP)[:, None] // NP ==
            jnp.arange(HEADS)[None, :]).astype(f32)

    grid = (STEPS,)
    out = pl.pallas_call(
        _tc_body,
        grid=grid,
        in_specs=[
            pl.BlockSpec((VSUB * R, F_IN), lambda i: (i, 0)),
            pl.BlockSpec((VSUB * R, NP), lambda i: (i, 0)),
            _full((F_IN, HEADS * HID)),
            _full((F_IN, 2 * HEADS)),
            _full((1, HID)),
            _full((HID, HEADS * HID)),
            _full((HID, 2 * HEADS)),
            _full((1, HID)),
            _full((N * HID, HID // 2)),
            _full((1, HID // 2)),
            _full((1, HID // 2)),
            _full((1, 1)),
            _full((R, R)),
            _full((R, R)),
            _full((R, 8 * NP)),
            _full((8 * NP, HEADS * R)),
            _full((HEADS, 8 * NP)),
            _full((NP, 8 * NP)),
            _full((8 * NP, HEADS)),
            _full((R, HEADS * R)),
        ],
        out_specs=pl.BlockSpec((VSUB * G, 1), lambda i: (i, 0)),
        out_shape=jax.ShapeDtypeStruct((B, 1), f32),
        compiler_params=pltpu.CompilerParams(
            dimension_semantics=("arbitrary",)),
    )(xp, cnt_rows, W1.astype(bf16), aw1, b1.reshape(1, HID),
      W2.astype(bf16), aw2, b2.reshape(1, HID), Wl.astype(bf16),
      bl.reshape(1, HID // 2), Wp.reshape(1, HID // 2), bp.reshape(1, 1),
      bsum, permb, msrc, t16big, tileh, cexph, sumh, bmask8)
    return out
